# baseline (speedup 1.0000x reference)
import functools

import jax
import jax.numpy as jnp
from jax import lax
from jax.experimental import pallas as pl
from jax.experimental.pallas import tpu as pltpu
from jax.experimental.pallas import tpu_sc as plsc

D = 64
S = 200
L = 16
NW = 32
EPS = 1e-5


def _rsqrt_newton(x):
    i = plsc.bitcast(x, jnp.int32)
    i = jnp.int32(0x5F3759DF) - (i >> 1)
    y = plsc.bitcast(i, jnp.float32)
    for _ in range(3):
        y = y * (1.5 - 0.5 * x * y * y)
    return y


def _emb_ln_body(ids_hbm, table_hbm, pe_hbm, scale_hbm, offset_hbm, out_hbm,
                 idx_v, rows_v, pe_v, scale_v, offset_v, gsem):
    nseq_w = ids_hbm.shape[0] // 2 // NW
    wid = lax.axis_index("s") * 2 + lax.axis_index("c")

    pltpu.sync_copy(pe_hbm, pe_v)
    pltpu.sync_copy(scale_hbm, scale_v)
    pltpu.sync_copy(offset_hbm, offset_v)

    def seq_body(j, carry):
        seq = wid * nseq_w + j
        pltpu.sync_copy(ids_hbm.at[pl.ds(seq * 2, 2)], idx_v)
        ga = pltpu.async_copy(table_hbm.at[idx_v.at[0]],
                              rows_v.at[pl.ds(0, 100)], gsem)
        gb = pltpu.async_copy(table_hbm.at[idx_v.at[1]],
                              rows_v.at[pl.ds(100, 100)], gsem)
        ga.wait()
        gb.wait()

        @plsc.parallel_loop(0, S, unroll=4)
        def row_body(r):
            e = []
            for k in range(4):
                t = rows_v[r, pl.ds(k * L, L)]
                p = pe_v[r, pl.ds(k * L, L)]
                e.append(t + p)
            s = (e[0] + e[1]) + (e[2] + e[3])
            q = (e[0] * e[0] + e[1] * e[1]) + (e[2] * e[2] + e[3] * e[3])
            ssum = jnp.sum(s)
            qsum = jnp.sum(q)
            mean = ssum * (1.0 / D)
            var = qsum * (1.0 / D) - mean * mean
            rstd = _rsqrt_newton(jnp.broadcast_to(var + EPS, (L,)))
            mean_v = jnp.broadcast_to(mean, (L,))
            for k in range(4):
                sc = scale_v[pl.ds(k * L, L)]
                of = offset_v[pl.ds(k * L, L)]
                rows_v[r, pl.ds(k * L, L)] = (e[k] - mean_v) * rstd * sc + of

        pltpu.sync_copy(rows_v, out_hbm.at[pl.ds(seq * S, S)])
        return carry

    lax.fori_loop(0, nseq_w, seq_body, jnp.int32(0))


@functools.partial(jax.jit, static_argnums=())
def _emb_ln(ids2, W_emb, pe, ln_scale, ln_offset):
    n_rows = ids2.shape[0] * 100
    mesh = plsc.VectorSubcoreMesh(core_axis_name="c", subcore_axis_name="s")
    f = pl.kernel(
        _emb_ln_body,
        out_type=jax.ShapeDtypeStruct((n_rows, D), jnp.float32),
        mesh=mesh,
        compiler_params=pltpu.CompilerParams(
            needs_layout_passes=False, use_tc_tiling_on_sc=False),
        scratch_types=[
            pltpu.VMEM((2, 100), jnp.int32),
            pltpu.VMEM((S, D), jnp.float32),
            pltpu.VMEM((S, D), jnp.float32),
            pltpu.VMEM((D,), jnp.float32),
            pltpu.VMEM((D,), jnp.float32),
            pltpu.SemaphoreType.DMA,
        ],
    )
    return f(ids2, W_emb, pe, ln_scale, ln_offset)


def kernel(token_ids, W_emb, pos_emb, ln_scale, ln_offset):
    B, seq = token_ids.shape
    ids2 = token_ids.reshape(B * seq // 100, 100).astype(jnp.int32)
    pe = pos_emb[:seq]
    out = _emb_ln(ids2, W_emb, pe, ln_scale, ln_offset)
    return out.reshape(B, seq, D)


# speedup vs baseline: 1.0980x; 1.0980x over previous
import functools

import jax
import jax.numpy as jnp
from jax import lax
from jax.experimental import pallas as pl
from jax.experimental.pallas import tpu as pltpu
from jax.experimental.pallas import tpu_sc as plsc

D = 64
S = 200
L = 16
NW = 32
NSEQ_W = 128
EPS = 1e-5


def _rsqrt_newton(x):
    i = plsc.bitcast(x, jnp.int32)
    i = jnp.int32(0x5F3759DF) - (i >> 1)
    y = plsc.bitcast(i, jnp.float32)
    for _ in range(2):
        y = y * (1.5 - 0.5 * x * y * y)
    return y


def _emb_ln_body(ids_hbm, table_hbm, pe_hbm, scale_hbm, offset_hbm, out_hbm,
                 idx_v, rows0, rows1, pe_v, scale_v, offset_v,
                 gsem0, gsem1, osem0, osem1):
    wid = lax.axis_index("s") * 2 + lax.axis_index("c")
    rows = (rows0, rows1)
    gsems = (gsem0, gsem1)
    osems = (osem0, osem1)

    pltpu.sync_copy(ids_hbm.at[pl.ds(wid * 2 * NSEQ_W, 2 * NSEQ_W)], idx_v)
    pltpu.sync_copy(pe_hbm, pe_v)
    pltpu.sync_copy(scale_hbm, scale_v)
    pltpu.sync_copy(offset_hbm, offset_v)

    def g_start(j, rb, sem):
        pltpu.async_copy(table_hbm.at[idx_v.at[2 * j]],
                         rb.at[pl.ds(0, 100)], sem)
        pltpu.async_copy(table_hbm.at[idx_v.at[2 * j + 1]],
                         rb.at[pl.ds(100, 100)], sem)

    def g_wait(rb, sem):
        pltpu.make_async_copy(table_hbm.at[idx_v.at[0]],
                              rb.at[pl.ds(0, 100)], sem).wait()
        pltpu.make_async_copy(table_hbm.at[idx_v.at[0]],
                              rb.at[pl.ds(100, 100)], sem).wait()

    def o_start(j, rb, sem):
        pltpu.async_copy(rb, out_hbm.at[pl.ds((wid * NSEQ_W + j) * S, S)], sem)

    def o_wait(rb, sem):
        pltpu.make_async_copy(rb, out_hbm.at[pl.ds(0, S)], sem).wait()

    def compute(rb):
        @plsc.parallel_loop(0, S, unroll=8)
        def row_body(r):
            e = []
            for k in range(4):
                t = rb[r, pl.ds(k * L, L)]
                p = pe_v[r, pl.ds(k * L, L)]
                e.append(t + p)
            s = (e[0] + e[1]) + (e[2] + e[3])
            q = (e[0] * e[0] + e[1] * e[1]) + (e[2] * e[2] + e[3] * e[3])
            mean = jnp.sum(s) * (1.0 / D)
            var = jnp.sum(q) * (1.0 / D) - mean * mean
            rstd = _rsqrt_newton(jnp.broadcast_to(var + EPS, (L,)))
            mean_v = jnp.broadcast_to(mean, (L,))
            for k in range(4):
                sc = scale_v[pl.ds(k * L, L)]
                of = offset_v[pl.ds(k * L, L)]
                rb[r, pl.ds(k * L, L)] = (e[k] - mean_v) * rstd * sc + of

    g_start(0, rows[0], gsems[0])

    def pair_body(g, carry):
        for b in (0, 1):
            j = 2 * g + b
            g_wait(rows[b], gsems[b])
            if b == 0:
                @pl.when(g >= 1)
                def _():
                    o_wait(rows[1], osems[1])
            else:
                o_wait(rows[0], osems[0])

            if b == 0:
                g_start(j + 1, rows[1], gsems[1])
            else:
                @pl.when(g < NSEQ_W // 2 - 1)
                def _():
                    g_start(j + 1, rows[0], gsems[0])

            compute(rows[b])
            o_start(j, rows[b], osems[b])
        return carry

    lax.fori_loop(0, NSEQ_W // 2, pair_body, jnp.int32(0))
    o_wait(rows[1], osems[1])


@jax.jit
def _emb_ln(ids2, W_emb, pe, ln_scale, ln_offset):
    n_rows = ids2.shape[0] * 100
    mesh = plsc.VectorSubcoreMesh(core_axis_name="c", subcore_axis_name="s")
    f = pl.kernel(
        _emb_ln_body,
        out_type=jax.ShapeDtypeStruct((n_rows, D), jnp.float32),
        mesh=mesh,
        compiler_params=pltpu.CompilerParams(
            needs_layout_passes=False, use_tc_tiling_on_sc=False),
        scratch_types=[
            pltpu.VMEM((2 * NSEQ_W, 100), jnp.int32),
            pltpu.VMEM((S, D), jnp.float32),
            pltpu.VMEM((S, D), jnp.float32),
            pltpu.VMEM((S, D), jnp.float32),
            pltpu.VMEM((D,), jnp.float32),
            pltpu.VMEM((D,), jnp.float32),
            pltpu.SemaphoreType.DMA,
            pltpu.SemaphoreType.DMA,
            pltpu.SemaphoreType.DMA,
            pltpu.SemaphoreType.DMA,
        ],
    )
    return f(ids2, W_emb, pe, ln_scale, ln_offset)


def kernel(token_ids, W_emb, pos_emb, ln_scale, ln_offset):
    B, seq = token_ids.shape
    ids2 = token_ids.reshape(B * seq // 100, 100).astype(jnp.int32)
    pe = pos_emb[:seq]
    out = _emb_ln(ids2, W_emb, pe, ln_scale, ln_offset)
    return out.reshape(B, seq, D)
